# baseline (device time: 70668 ns/iter reference)
import jax
import jax.numpy as jnp
from jax import lax
from jax.experimental import pallas as pl
from jax.experimental.pallas import tpu as pltpu

N_DEV = 4
B = 2
SQ = 512
DM = 768
HQ = 8
DH = 64
HD = HQ * DH
SKV = 512
BLK = 64


def kernel(x, Wq, K_ext, V_ext, Wo):
    K2 = K_ext.reshape(B, SKV, HD)
    V2 = V_ext.reshape(B, SKV, HD)

    def body(x_ref, wq_ref, k_ref, v_ref, wo_ref, out_ref,
             part_ref, den_ref, cnum_ref, cden_ref,
             snum, sden, rnum, rden):
        my = lax.axis_index("i")

        bsem = pltpu.get_barrier_semaphore()
        for d in range(1, N_DEV):
            peer = lax.rem(my + d, N_DEV)
            pl.semaphore_signal(
                bsem, inc=1,
                device_id=(peer,), device_id_type=pl.DeviceIdType.MESH,
            )
        pl.semaphore_wait(bsem, N_DEV - 1)

        qi = lax.broadcasted_iota(jnp.int32, (SQ, SKV), 0)
        kj = lax.broadcasted_iota(jnp.int32, (SQ, SKV), 1)
        mask = ((qi // BLK) % 4) == ((kj // BLK) % 4)

        for b in range(B):
            q_b = jnp.dot(x_ref[b], wq_ref[...],
                          preferred_element_type=jnp.float32)
            for h in range(HQ):
                sl = pl.ds(h * DH, DH)
                qh = q_b[:, h * DH:(h + 1) * DH]
                kh = k_ref[b, :, sl]
                s = lax.dot_general(
                    qh, kh, (((1,), (1,)), ((), ())),
                    preferred_element_type=jnp.float32) * 0.125
                w = jnp.where(mask, jnp.exp(s), 0.0)
                part_ref[b, :, sl] = jnp.dot(
                    w, v_ref[b, :, sl], preferred_element_type=jnp.float32)
                den_ref[:, b * HQ + h:b * HQ + h + 1] = jnp.sum(
                    w, axis=1, keepdims=True)

        rdmas = []
        for d in range(1, N_DEV):
            j = d - 1
            peer = lax.rem(my + d, N_DEV)
            r_num = pltpu.make_async_remote_copy(
                src_ref=part_ref, dst_ref=cnum_ref.at[j],
                send_sem=snum.at[j], recv_sem=rnum.at[j],
                device_id=(peer,), device_id_type=pl.DeviceIdType.MESH,
            )
            r_den = pltpu.make_async_remote_copy(
                src_ref=den_ref, dst_ref=cden_ref.at[j],
                send_sem=sden.at[j], recv_sem=rden.at[j],
                device_id=(peer,), device_id_type=pl.DeviceIdType.MESH,
            )
            r_num.start()
            r_den.start()
            rdmas.append((r_num, r_den))
        for r_num, r_den in rdmas:
            r_num.wait()
            r_den.wait()

        den_tot = (den_ref[...] + cden_ref[0] + cden_ref[1] + cden_ref[2])
        for b in range(B):
            for h in range(HQ):
                sl = pl.ds(h * DH, DH)
                num_h = (part_ref[b, :, sl] + cnum_ref[0, b, :, sl]
                         + cnum_ref[1, b, :, sl] + cnum_ref[2, b, :, sl])
                c = b * HQ + h
                part_ref[b, :, sl] = num_h / den_tot[:, c:c + 1]
            out_ref[b] = jnp.dot(part_ref[b], wo_ref[...],
                                 preferred_element_type=jnp.float32)

    return pl.pallas_call(
        body,
        out_shape=jax.ShapeDtypeStruct((B, SQ, DM), jnp.float32),
        in_specs=[pl.BlockSpec(memory_space=pltpu.VMEM)] * 5,
        out_specs=pl.BlockSpec(memory_space=pltpu.VMEM),
        scratch_shapes=[
            pltpu.VMEM((B, SQ, HD), jnp.float32),
            pltpu.VMEM((SQ, B * HQ), jnp.float32),
            pltpu.VMEM((3, B, SQ, HD), jnp.float32),
            pltpu.VMEM((3, SQ, B * HQ), jnp.float32),
            pltpu.SemaphoreType.DMA((3,)),
            pltpu.SemaphoreType.DMA((3,)),
            pltpu.SemaphoreType.DMA((3,)),
            pltpu.SemaphoreType.DMA((3,)),
        ],
        compiler_params=pltpu.CompilerParams(collective_id=0),
    )(x, Wq, K2, V2, Wo)


# device time: 47689 ns/iter; 1.4819x vs baseline; 1.4819x over previous
import jax
import jax.numpy as jnp
from jax import lax
from jax.experimental import pallas as pl
from jax.experimental.pallas import tpu as pltpu

N_DEV = 4
B = 2
SQ = 512
DM = 768
HQ = 8
DH = 64
HD = HQ * DH
SKV = 512
BLK = 64
NRES = 4


def kernel(x, Wq, K_ext, V_ext, Wo):
    K2 = K_ext.reshape(B, SKV, HD)
    V2 = V_ext.reshape(B, SKV, HD)

    def body(x_ref, wq_ref, k_ref, v_ref, wo_ref, out_ref,
             part_ref, den_ref, send_ref, cnum_ref, cden_ref,
             snum, sden, rnum, rden):
        my = lax.axis_index("i")

        bsem = pltpu.get_barrier_semaphore()
        for d in range(1, N_DEV):
            peer = lax.rem(my + d, N_DEV)
            pl.semaphore_signal(
                bsem, inc=1,
                device_id=(peer,), device_id_type=pl.DeviceIdType.MESH,
            )
        pl.semaphore_wait(bsem, N_DEV - 1)

        wq_b = wq_ref[...].astype(jnp.bfloat16)

        num_rdmas = []
        for b in range(B):
            q_b = jnp.dot(x_ref[b].astype(jnp.bfloat16), wq_b,
                          preferred_element_type=jnp.float32
                          ).astype(jnp.bfloat16)
            for h in range(HQ):
                sl = pl.ds(h * DH, DH)
                kh = k_ref[b, :, sl].astype(jnp.bfloat16)
                vh = v_ref[b, :, sl].astype(jnp.bfloat16)
                qh = q_b[:, h * DH:(h + 1) * DH]
                for r in range(NRES):
                    lo = slice(BLK * r, BLK * (r + 1))
                    hi = slice(BLK * (r + 4), BLK * (r + 5))
                    r0 = pl.ds(BLK * r, BLK)
                    r1 = pl.ds(BLK * (r + 4), BLK)
                    qch = jnp.concatenate([qh[lo, :], qh[hi, :]], axis=0)
                    kch = jnp.concatenate([kh[lo, :], kh[hi, :]], axis=0)
                    vch = jnp.concatenate([vh[lo, :], vh[hi, :]], axis=0)
                    s = lax.dot_general(
                        qch, kch, (((1,), (1,)), ((), ())),
                        preferred_element_type=jnp.float32) * 0.125
                    w = jnp.exp(s)
                    numch = jnp.dot(w.astype(jnp.bfloat16), vch,
                                    preferred_element_type=jnp.float32)
                    dch = jnp.sum(w, axis=1, keepdims=True)
                    part_ref[b, r0, sl] = numch[:BLK]
                    part_ref[b, r1, sl] = numch[BLK:]
                    send_ref[b, r0, sl] = numch[:BLK].astype(jnp.bfloat16)
                    send_ref[b, r1, sl] = numch[BLK:].astype(jnp.bfloat16)
                    c = b * HQ + h
                    den_ref[r0, c:c + 1] = dch[:BLK]
                    den_ref[r1, c:c + 1] = dch[BLK:]

            for d in range(1, N_DEV):
                j = d - 1
                peer = lax.rem(my + d, N_DEV)
                r_num = pltpu.make_async_remote_copy(
                    src_ref=send_ref.at[b], dst_ref=cnum_ref.at[j, b],
                    send_sem=snum.at[j, b], recv_sem=rnum.at[j, b],
                    device_id=(peer,), device_id_type=pl.DeviceIdType.MESH,
                )
                r_num.start()
                num_rdmas.append(r_num)

        den_rdmas = []
        for d in range(1, N_DEV):
            j = d - 1
            peer = lax.rem(my + d, N_DEV)
            r_den = pltpu.make_async_remote_copy(
                src_ref=den_ref, dst_ref=cden_ref.at[j],
                send_sem=sden.at[j], recv_sem=rden.at[j],
                device_id=(peer,), device_id_type=pl.DeviceIdType.MESH,
            )
            r_den.start()
            den_rdmas.append(r_den)

        for r in num_rdmas + den_rdmas:
            r.wait()

        den_tot = (den_ref[...] + cden_ref[0] + cden_ref[1] + cden_ref[2])
        wo_b = wo_ref[...].astype(jnp.bfloat16)
        for b in range(B):
            for h in range(HQ):
                sl = pl.ds(h * DH, DH)
                num_h = (part_ref[b, :, sl]
                         + cnum_ref[0, b, :, sl].astype(jnp.float32)
                         + cnum_ref[1, b, :, sl].astype(jnp.float32)
                         + cnum_ref[2, b, :, sl].astype(jnp.float32))
                c = b * HQ + h
                send_ref[b, :, sl] = (
                    num_h / den_tot[:, c:c + 1]).astype(jnp.bfloat16)
            out_ref[b] = jnp.dot(send_ref[b], wo_b,
                                 preferred_element_type=jnp.float32)

    return pl.pallas_call(
        body,
        out_shape=jax.ShapeDtypeStruct((B, SQ, DM), jnp.float32),
        in_specs=[pl.BlockSpec(memory_space=pltpu.VMEM)] * 5,
        out_specs=pl.BlockSpec(memory_space=pltpu.VMEM),
        scratch_shapes=[
            pltpu.VMEM((B, SQ, HD), jnp.float32),
            pltpu.VMEM((SQ, B * HQ), jnp.float32),
            pltpu.VMEM((B, SQ, HD), jnp.bfloat16),
            pltpu.VMEM((3, B, SQ, HD), jnp.bfloat16),
            pltpu.VMEM((3, SQ, B * HQ), jnp.float32),
            pltpu.SemaphoreType.DMA((3, B)),
            pltpu.SemaphoreType.DMA((3,)),
            pltpu.SemaphoreType.DMA((3, B)),
            pltpu.SemaphoreType.DMA((3,)),
        ],
        compiler_params=pltpu.CompilerParams(collective_id=0),
    )(x, Wq, K2, V2, Wo)


# device time: 32948 ns/iter; 2.1448x vs baseline; 1.4474x over previous
import jax
import jax.numpy as jnp
from jax import lax
from jax.experimental import pallas as pl
from jax.experimental.pallas import tpu as pltpu

N_DEV = 4
B = 2
SQ = 512
DM = 768
HQ = 8
DH = 64
HD = HQ * DH
SKV = 512
BLK = 64
NRES = 4
QTR = SQ // N_DEV


def kernel(x, Wq, K_ext, V_ext, Wo):
    K2 = K_ext.reshape(B, SKV, HD)
    V2 = V_ext.reshape(B, SKV, HD)

    def body(x_ref, wq_ref, k_ref, v_ref, wo_ref, out_ref,
             den_ref, send_ref, xp_ref, kp_ref, vp_ref,
             rsn_ref, rsd_ref,
             s_rs, s_rsd, s_ag, r_rs, r_rsd, r_ag):
        my = lax.axis_index("i")
        myq = pl.ds(my * QTR, QTR)

        bsem = pltpu.get_barrier_semaphore()
        for d in range(1, N_DEV):
            peer = lax.rem(my + d, N_DEV)
            pl.semaphore_signal(
                bsem, inc=1,
                device_id=(peer,), device_id_type=pl.DeviceIdType.MESH,
            )

        wq_b = (wq_ref[...] * 0.125).astype(jnp.bfloat16)
        ones_bd = (lax.broadcasted_iota(jnp.int32, (HQ * 2 * BLK, HQ), 0)
                   // (2 * BLK)
                   == lax.broadcasted_iota(jnp.int32, (HQ * 2 * BLK, HQ), 1)
                   ).astype(jnp.bfloat16)

        rs_sends = []
        for b in range(B):
            for g in range(NRES):
                n0, n1 = pl.ds(128 * g, BLK), pl.ds(128 * g + BLK, BLK)
                o0, o1 = pl.ds(BLK * g, BLK), pl.ds(BLK * (g + 4), BLK)
                xp_ref[n0, :] = x_ref[b, o0, :].astype(jnp.bfloat16)
                xp_ref[n1, :] = x_ref[b, o1, :].astype(jnp.bfloat16)
                kp_ref[n0, :] = k_ref[b, o0, :].astype(jnp.bfloat16)
                kp_ref[n1, :] = k_ref[b, o1, :].astype(jnp.bfloat16)
                vp_ref[n0, :] = v_ref[b, o0, :].astype(jnp.bfloat16)
                vp_ref[n1, :] = v_ref[b, o1, :].astype(jnp.bfloat16)

            qp = jnp.dot(xp_ref[...], wq_b,
                         preferred_element_type=jnp.float32
                         ).astype(jnp.bfloat16)

            if b == 0:
                pl.semaphore_wait(bsem, N_DEV - 1)

            for r in range(NRES):
                rs = pl.ds(128 * r, 128)
                w_list = []
                for h in range(HQ):
                    sl = pl.ds(h * DH, DH)
                    s = lax.dot_general(
                        qp[128 * r:128 * (r + 1), h * DH:(h + 1) * DH],
                        kp_ref[rs, sl],
                        (((1,), (1,)), ((), ())),
                        preferred_element_type=jnp.float32)
                    w = jnp.exp(s).astype(jnp.bfloat16)
                    numch = jnp.dot(w, vp_ref[rs, sl],
                                    preferred_element_type=jnp.float32)
                    send_ref[b, rs, sl] = numch.astype(jnp.bfloat16)
                    w_list.append(w)
                wcat = jnp.concatenate(w_list, axis=1)
                den_blk = jnp.dot(wcat, ones_bd,
                                  preferred_element_type=jnp.float32)
                den_ref[rs, b * HQ:(b + 1) * HQ] = den_blk

                r_n = pltpu.make_async_remote_copy(
                    src_ref=send_ref.at[b, rs], dst_ref=rsn_ref.at[my, b],
                    send_sem=s_rs.at[r, b], recv_sem=r_rs.at[my, b],
                    device_id=(r,), device_id_type=pl.DeviceIdType.MESH,
                )
                r_d = pltpu.make_async_remote_copy(
                    src_ref=den_ref.at[rs], dst_ref=rsd_ref.at[my, b],
                    send_sem=s_rsd.at[r, b], recv_sem=r_rsd.at[my, b],
                    device_id=(r,), device_id_type=pl.DeviceIdType.MESH,
                )

                @pl.when(my != r)
                def _(r_n=r_n, r_d=r_d):
                    r_n.start()
                    r_d.start()

                @pl.when(my == r)
                def _(b=b, r=r, rs=rs):
                    rsn_ref[r, b] = send_ref[b, rs, :]
                    rsd_ref[r, b] = den_ref[rs, :]

                rs_sends.append((r, r_n, r_d))

        wo_b = wo_ref[...].astype(jnp.bfloat16)
        ag_rdmas = [[], []]
        for b in range(B):
            for s in range(N_DEV):
                r_n = pltpu.make_async_remote_copy(
                    src_ref=rsn_ref.at[s, b], dst_ref=rsn_ref.at[s, b],
                    send_sem=s_rs.at[s, b], recv_sem=r_rs.at[s, b],
                    device_id=(my,), device_id_type=pl.DeviceIdType.MESH,
                )
                r_d = pltpu.make_async_remote_copy(
                    src_ref=rsd_ref.at[s, b], dst_ref=rsd_ref.at[s, b],
                    send_sem=s_rsd.at[s, b], recv_sem=r_rsd.at[s, b],
                    device_id=(my,), device_id_type=pl.DeviceIdType.MESH,
                )

                @pl.when(s != my)
                def _(r_n=r_n, r_d=r_d):
                    r_n.wait_recv()
                    r_d.wait_recv()

            bc = slice(b * HQ, (b + 1) * HQ)
            den_q = (rsd_ref[0, b, :, bc] + rsd_ref[1, b, :, bc]
                     + rsd_ref[2, b, :, bc] + rsd_ref[3, b, :, bc])
            for h in range(HQ):
                sl = pl.ds(h * DH, DH)
                num_q = (rsn_ref[0, b, :, sl].astype(jnp.float32)
                         + rsn_ref[1, b, :, sl].astype(jnp.float32)
                         + rsn_ref[2, b, :, sl].astype(jnp.float32)
                         + rsn_ref[3, b, :, sl].astype(jnp.float32))
                send_ref[b, myq, sl] = (
                    num_q / den_q[:, h:h + 1]).astype(jnp.bfloat16)
            for d in range(1, N_DEV):
                j = d - 1
                peer = lax.rem(my + d, N_DEV)
                r_a = pltpu.make_async_remote_copy(
                    src_ref=send_ref.at[b, myq], dst_ref=send_ref.at[b, myq],
                    send_sem=s_ag.at[j, b], recv_sem=r_ag.at[j, b],
                    device_id=(peer,), device_id_type=pl.DeviceIdType.MESH,
                )
                r_a.start()
                ag_rdmas[b].append(r_a)

        for b in range(B):
            def project(q, ctx_q):
                o_q = jnp.dot(ctx_q, wo_b,
                              preferred_element_type=jnp.float32)
                out_ref[b, pl.ds(BLK * q, BLK), :] = o_q[:BLK]
                out_ref[b, pl.ds(BLK * q + NRES * BLK, BLK), :] = o_q[BLK:]

            project(my, send_ref[b, myq, :])
            for j, r_a in enumerate(ag_rdmas[b]):
                r_a.wait()
                qj = lax.rem(my - j - 1 + N_DEV, N_DEV)
                project(qj, send_ref[b, pl.ds(qj * QTR, QTR), :])

        for r, r_n, r_d in rs_sends:
            @pl.when(my != r)
            def _(r_n=r_n, r_d=r_d):
                r_n.wait_send()
                r_d.wait_send()

    return pl.pallas_call(
        body,
        out_shape=jax.ShapeDtypeStruct((B, SQ, DM), jnp.float32),
        in_specs=[pl.BlockSpec(memory_space=pltpu.VMEM)] * 5,
        out_specs=pl.BlockSpec(memory_space=pltpu.VMEM),
        scratch_shapes=[
            pltpu.VMEM((SQ, B * HQ), jnp.float32),
            pltpu.VMEM((B, SQ, HD), jnp.bfloat16),
            pltpu.VMEM((SQ, DM), jnp.bfloat16),
            pltpu.VMEM((SQ, HD), jnp.bfloat16),
            pltpu.VMEM((SQ, HD), jnp.bfloat16),
            pltpu.VMEM((N_DEV, B, QTR, HD), jnp.bfloat16),
            pltpu.VMEM((N_DEV, B, QTR, B * HQ), jnp.float32),
            pltpu.SemaphoreType.DMA((N_DEV, B)),
            pltpu.SemaphoreType.DMA((N_DEV, B)),
            pltpu.SemaphoreType.DMA((3, B)),
            pltpu.SemaphoreType.DMA((N_DEV, B)),
            pltpu.SemaphoreType.DMA((N_DEV, B)),
            pltpu.SemaphoreType.DMA((3, B)),
        ],
        compiler_params=pltpu.CompilerParams(collective_id=0),
    )(x, Wq, K2, V2, Wo)


# device time: 32017 ns/iter; 2.2072x vs baseline; 1.0291x over previous
import jax
import jax.numpy as jnp
from jax import lax
from jax.experimental import pallas as pl
from jax.experimental.pallas import tpu as pltpu

N_DEV = 4
B = 2
SQ = 512
DM = 768
HQ = 8
DH = 64
HD = HQ * DH
SKV = 512
BLK = 64
NRES = 4
QTR = SQ // N_DEV


def kernel(x, Wq, K_ext, V_ext, Wo):
    K2 = K_ext.reshape(B, SKV, HD)
    V2 = V_ext.reshape(B, SKV, HD)

    def body(x_ref, wq_ref, k_ref, v_ref, wo_ref, out_ref,
             den_ref, send_ref, xp_ref, kp_ref, vp_ref,
             rsn_ref, rsd_ref,
             s_rs, s_rsd, s_ag, r_rs, r_rsd, r_ag):
        my = lax.axis_index("i")
        myq = pl.ds(my * QTR, QTR)

        bsem = pltpu.get_barrier_semaphore()
        for d in range(1, N_DEV):
            peer = lax.rem(my + d, N_DEV)
            pl.semaphore_signal(
                bsem, inc=1,
                device_id=(peer,), device_id_type=pl.DeviceIdType.MESH,
            )

        wq_b = (wq_ref[...] * 0.125).astype(jnp.bfloat16)
        ones_bd = (lax.broadcasted_iota(jnp.int32, (HQ * 2 * BLK, HQ), 0)
                   // (2 * BLK)
                   == lax.broadcasted_iota(jnp.int32, (HQ * 2 * BLK, HQ), 1)
                   ).astype(jnp.bfloat16)

        rs_by_b = [[], []]
        for b in range(B):
            for g in range(NRES):
                n0, n1 = pl.ds(128 * g, BLK), pl.ds(128 * g + BLK, BLK)
                o0, o1 = pl.ds(BLK * g, BLK), pl.ds(BLK * (g + 4), BLK)
                xp_ref[n0, :] = x_ref[b, o0, :].astype(jnp.bfloat16)
                xp_ref[n1, :] = x_ref[b, o1, :].astype(jnp.bfloat16)
                kp_ref[n0, :] = k_ref[b, o0, :].astype(jnp.bfloat16)
                kp_ref[n1, :] = k_ref[b, o1, :].astype(jnp.bfloat16)
                vp_ref[n0, :] = v_ref[b, o0, :].astype(jnp.bfloat16)
                vp_ref[n1, :] = v_ref[b, o1, :].astype(jnp.bfloat16)

            qp = jnp.dot(xp_ref[...], wq_b,
                         preferred_element_type=jnp.float32
                         ).astype(jnp.bfloat16)

            for r in range(NRES):
                rs = pl.ds(128 * r, 128)
                w_list = []
                for h in range(HQ):
                    sl = pl.ds(h * DH, DH)
                    s = lax.dot_general(
                        qp[128 * r:128 * (r + 1), h * DH:(h + 1) * DH],
                        kp_ref[rs, sl],
                        (((1,), (1,)), ((), ())),
                        preferred_element_type=jnp.float32)
                    w = jnp.exp(s).astype(jnp.bfloat16)
                    numch = jnp.dot(w, vp_ref[rs, sl],
                                    preferred_element_type=jnp.float32)
                    send_ref[b, rs, sl] = numch.astype(jnp.bfloat16)
                    w_list.append(w)
                wcat = jnp.concatenate(w_list, axis=1)
                den_blk = jnp.dot(wcat, ones_bd,
                                  preferred_element_type=jnp.float32)
                den_ref[rs, b * HQ:(b + 1) * HQ] = den_blk

            if b == 0:
                pl.semaphore_wait(bsem, N_DEV - 1)
            for d in range(1, N_DEV):
                j = d - 1
                peer = lax.rem(my + d, N_DEV)
                pq = pl.ds(peer * QTR, QTR)
                r_n = pltpu.make_async_remote_copy(
                    src_ref=send_ref.at[b, pq], dst_ref=rsn_ref.at[j, b],
                    send_sem=s_rs.at[j, b], recv_sem=r_rs.at[j, b],
                    device_id=(peer,), device_id_type=pl.DeviceIdType.MESH,
                )
                r_d = pltpu.make_async_remote_copy(
                    src_ref=den_ref.at[pq], dst_ref=rsd_ref.at[j, b],
                    send_sem=s_rsd.at[j, b], recv_sem=r_rsd.at[j, b],
                    device_id=(peer,), device_id_type=pl.DeviceIdType.MESH,
                )
                r_n.start()
                r_d.start()
                rs_by_b[b] += [r_n, r_d]

        wo_b = wo_ref[...].astype(jnp.bfloat16)
        ag_rdmas = [[], []]
        for b in range(B):
            for r_n in rs_by_b[b]:
                r_n.wait()
            den_q = (den_ref[myq, b * HQ:(b + 1) * HQ]
                     + rsd_ref[0, b, :, b * HQ:(b + 1) * HQ]
                     + rsd_ref[1, b, :, b * HQ:(b + 1) * HQ]
                     + rsd_ref[2, b, :, b * HQ:(b + 1) * HQ])
            rec_q = 1.0 / den_q
            for h in range(HQ):
                sl = pl.ds(h * DH, DH)
                num_q = (send_ref[b, myq, sl].astype(jnp.float32)
                         + rsn_ref[0, b, :, sl].astype(jnp.float32)
                         + rsn_ref[1, b, :, sl].astype(jnp.float32)
                         + rsn_ref[2, b, :, sl].astype(jnp.float32))
                send_ref[b, myq, sl] = (
                    num_q * rec_q[:, h:h + 1]).astype(jnp.bfloat16)
            for d in range(1, N_DEV):
                j = d - 1
                peer = lax.rem(my + d, N_DEV)
                r_a = pltpu.make_async_remote_copy(
                    src_ref=send_ref.at[b, myq], dst_ref=send_ref.at[b, myq],
                    send_sem=s_ag.at[j, b], recv_sem=r_ag.at[j, b],
                    device_id=(peer,), device_id_type=pl.DeviceIdType.MESH,
                )
                r_a.start()
                ag_rdmas[b].append(r_a)

        for b in range(B):
            def project(q, ctx_q):
                o_q = jnp.dot(ctx_q, wo_b,
                              preferred_element_type=jnp.float32)
                out_ref[b, pl.ds(BLK * q, BLK), :] = o_q[:BLK]
                out_ref[b, pl.ds(BLK * q + NRES * BLK, BLK), :] = o_q[BLK:]

            project(my, send_ref[b, myq, :])
            for j, r_a in enumerate(ag_rdmas[b]):
                r_a.wait()
                qj = lax.rem(my - j - 1 + N_DEV, N_DEV)
                project(qj, send_ref[b, pl.ds(qj * QTR, QTR), :])

    return pl.pallas_call(
        body,
        out_shape=jax.ShapeDtypeStruct((B, SQ, DM), jnp.float32),
        in_specs=[pl.BlockSpec(memory_space=pltpu.VMEM)] * 5,
        out_specs=pl.BlockSpec(memory_space=pltpu.VMEM),
        scratch_shapes=[
            pltpu.VMEM((SQ, B * HQ), jnp.float32),
            pltpu.VMEM((B, SQ, HD), jnp.bfloat16),
            pltpu.VMEM((SQ, DM), jnp.bfloat16),
            pltpu.VMEM((SQ, HD), jnp.bfloat16),
            pltpu.VMEM((SQ, HD), jnp.bfloat16),
            pltpu.VMEM((3, B, QTR, HD), jnp.bfloat16),
            pltpu.VMEM((3, B, QTR, B * HQ), jnp.float32),
            pltpu.SemaphoreType.DMA((3, B)),
            pltpu.SemaphoreType.DMA((3, B)),
            pltpu.SemaphoreType.DMA((3, B)),
            pltpu.SemaphoreType.DMA((3, B)),
            pltpu.SemaphoreType.DMA((3, B)),
            pltpu.SemaphoreType.DMA((3, B)),
        ],
        compiler_params=pltpu.CompilerParams(collective_id=0),
    )(x, Wq, K2, V2, Wo)
